# baseline (speedup 1.0000x reference)
import numpy as np

import jax
import jax.numpy as jnp
from jax.experimental import pallas as pl
from jax.experimental.pallas import tpu as pltpu

_C = 8
_L = 16
_N = 128
_ROUNDS = 7


def _tp(a, b):
    A, n = a.shape[-2], a.shape[-1]
    Bd = b.shape[-2]
    return (a[..., :, None, :] * b[..., None, :, :]).reshape(
        *a.shape[:-2], A * Bd, n)


def _exp1(d):
    e2 = _tp(d * 0.5, d)
    e3 = _tp(e2, d * (1.0 / 3.0))
    e4 = _tp(e3, d * 0.25)
    return (d, e2, e3, e4)


def _mul3(a, b):
    a1, a2, a3 = a[0], a[1], a[2]
    b1, b2, b3 = b[0], b[1], b[2]
    parts = []
    for m in range(_C):
        parts.append(a3[..., m * 64:(m + 1) * 64, :] + b3[..., m * 64:(m + 1) * 64, :]
                     + a1[..., m:m + 1, :] * b2
                     + _tp(a2[..., m * 8:(m + 1) * 8, :], b1))
    return jnp.concatenate(parts, axis=-2)


def _mul4(a, b):
    a1, a2, a3, a4 = a
    b1, b2, b3, b4 = b
    parts = []
    for m in range(_C):
        parts.append(a4[..., m * 512:(m + 1) * 512, :] + b4[..., m * 512:(m + 1) * 512, :]
                     + a1[..., m:m + 1, :] * b3
                     + _tp(a2[..., m * 8:(m + 1) * 8, :], b2)
                     + _tp(a3[..., m * 64:(m + 1) * 64, :], b1))
    return jnp.concatenate(parts, axis=-2)


def _ta_mul(a, b):
    c1 = a[0] + b[0]
    c2 = a[1] + b[1] + _tp(a[0], b[0])
    c3 = _mul3(a, b)
    c4 = _mul4(a, b)
    return (c1, c2, c3, c4)


def _chen_step(carry, d):
    a1, a2, a3, a4 = carry
    e2 = _tp(d * 0.5, d)
    e3 = _tp(e2, d * (1.0 / 3.0))
    dq = d * 0.25
    c1 = a1 + d
    c2 = a2 + e2 + _tp(a1, d)
    p3 = []
    p4 = []
    for m in range(_C):
        p3.append(a3[..., m * 64:(m + 1) * 64, :] + e3[..., m * 64:(m + 1) * 64, :]
                  + a1[..., m:m + 1, :] * e2
                  + _tp(a2[..., m * 8:(m + 1) * 8, :], d))
        p4.append(a4[..., m * 512:(m + 1) * 512, :]
                  + _tp(e3[..., m * 64:(m + 1) * 64, :], dq)
                  + a1[..., m:m + 1, :] * e3
                  + _tp(a2[..., m * 8:(m + 1) * 8, :], e2)
                  + _tp(a3[..., m * 64:(m + 1) * 64, :], d))
    return (c1, c2, jnp.concatenate(p3, axis=-2), jnp.concatenate(p4, axis=-2))


def _blamem_kernel(inc_ref, w1_ref, b1_ref, w2_ref, b2_ref, out_ref):
    carry0 = _exp1(inc_ref[:, 0])

    def chen_body(s, carry):
        d = inc_ref[:, s]
        return _chen_step(carry, d)

    sig = jax.lax.fori_loop(1, _L, chen_body, carry0)

    lane = jax.lax.broadcasted_iota(jnp.int32, (1, 1, _N), 2)

    def scan_body(i, pref):
        d = jax.lax.shift_left(jnp.int32(1), i)
        maskf = (lane >= d).astype(jnp.float32)
        shifted = tuple(pltpu.roll(lv, d, 2) * maskf for lv in pref)
        return _ta_mul(shifted, pref)

    s1, s2, s3, s4 = jax.lax.fori_loop(0, _ROUNDS, scan_body, sig)

    s1h = s1 * -0.5
    s2h = s2 * -0.5
    s3h = s3 * -0.5
    s1t = s1 * (1.0 / 3.0)
    s2t = s2 * (1.0 / 3.0)
    s1q = s1 * -0.25
    p2 = _tp(s1, s1)
    p3 = _tp(s1, s2) + _tp(s2, s1)
    q3 = _tp(s1, p2)
    l1 = s1
    l2 = s2 - 0.5 * p2
    l3 = s3 - 0.5 * p3 + (1.0 / 3.0) * q3
    l4_parts = []
    for m in range(_C):
        l4_parts.append(
            s4[..., m * 512:(m + 1) * 512, :]
            + s1h[..., m:m + 1, :] * s3
            + _tp(s2h[..., m * 8:(m + 1) * 8, :], s2)
            + _tp(s3h[..., m * 64:(m + 1) * 64, :], s1)
            + s1t[..., m:m + 1, :] * p3
            + _tp(s2t[..., m * 8:(m + 1) * 8, :], p2)
            + s1q[..., m:m + 1, :] * q3)
    l4 = jnp.concatenate(l4_parts, axis=-2)

    m1 = jnp.mean(l1, axis=-1, keepdims=True)
    m2 = jnp.mean(l2, axis=-1, keepdims=True)
    m3 = jnp.mean(l3, axis=-1, keepdims=True)
    m4 = jnp.mean(l4, axis=-1, keepdims=True)

    dn = (((1,), (0,)), ((), ()))
    h = (jax.lax.dot_general(m1, w1_ref[0:8, :], dn,
                             preferred_element_type=jnp.float32)
         + jax.lax.dot_general(m2, w1_ref[8:72, :], dn,
                               preferred_element_type=jnp.float32)
         + jax.lax.dot_general(m3, w1_ref[72:584, :], dn,
                               preferred_element_type=jnp.float32)
         + jax.lax.dot_general(m4, w1_ref[584:4680, :], dn,
                               preferred_element_type=jnp.float32)
         + b1_ref[...])
    h = jnp.maximum(h, 0.0)
    dn2 = (((2,), (0,)), ((), ()))
    out_ref[...] = (jax.lax.dot_general(h, w2_ref[...], dn2,
                                        preferred_element_type=jnp.float32)
                    + b2_ref[...])


def kernel(x, W1, b1, W2, b2):
    B, T, Cin = x.shape
    C = Cin + 1
    N = T // _L
    H = W1.shape[1]

    t = jnp.linspace(0.0, 1.0, T, dtype=x.dtype)
    path = jnp.concatenate(
        [x, jnp.broadcast_to(t[None, :, None], (B, T, 1)).astype(x.dtype)],
        axis=-1)
    inc = jnp.diff(path, axis=1, prepend=jnp.zeros((B, 1, C), x.dtype))
    inc_t = inc.reshape(B, N, _L, C).transpose(0, 2, 3, 1)

    b1_2d = b1.reshape(1, H)
    b2_2d = b2.reshape(1, 1)

    G = 2
    out = pl.pallas_call(
        _blamem_kernel,
        grid=(B // G,),
        in_specs=[
            pl.BlockSpec((G, _L, C, N), lambda b: (b, 0, 0, 0)),
            pl.BlockSpec(W1.shape, lambda b: (0, 0)),
            pl.BlockSpec((1, H), lambda b: (0, 0)),
            pl.BlockSpec(W2.shape, lambda b: (0, 0)),
            pl.BlockSpec((1, 1), lambda b: (0, 0)),
        ],
        out_specs=pl.BlockSpec((G, 1, 1), lambda b: (b, 0, 0)),
        out_shape=jax.ShapeDtypeStruct((B, 1, 1), jnp.float32),
        compiler_params=pltpu.CompilerParams(
            dimension_semantics=("arbitrary",),
            vmem_limit_bytes=56 * 1024 * 1024,
        ),
    )(inc_t, W1, b1_2d, W2, b2_2d)
    return out.reshape(B, 1)


# speedup vs baseline: 1.2303x; 1.2303x over previous
import numpy as np

import jax
import jax.numpy as jnp
from jax.experimental import pallas as pl
from jax.experimental.pallas import tpu as pltpu

_C = 8
_L = 16
_N = 128
_ROUNDS = 7


def _tp(a, b):
    A, n = a.shape
    Bd = b.shape[0]
    return (a[:, None, :] * b[None, :, :]).reshape(A * Bd, n)


def _exp1(d):
    e2 = _tp(d * 0.5, d)
    e3 = _tp(e2, d * (1.0 / 3.0))
    e4 = _tp(e3, d * 0.25)
    return (d, e2, e3, e4)


def _mul3(a, b):
    a1, a2, a3 = a[0], a[1], a[2]
    b1, b2, b3 = b[0], b[1], b[2]
    parts = []
    for m in range(_C):
        parts.append(a3[m * 64:(m + 1) * 64] + b3[m * 64:(m + 1) * 64]
                     + a1[m:m + 1] * b2
                     + _tp(a2[m * 8:(m + 1) * 8], b1))
    return jnp.concatenate(parts, axis=0)


def _mul4(a, b):
    a1, a2, a3, a4 = a
    b1, b2, b3, b4 = b
    parts = []
    for k in range(2 * _C):
        m, h = k >> 1, k & 1
        parts.append(a4[k * 256:(k + 1) * 256] + b4[k * 256:(k + 1) * 256]
                     + a1[m:m + 1] * b3[h * 256:(h + 1) * 256]
                     + _tp(a2[k * 4:(k + 1) * 4], b2)
                     + _tp(a3[k * 32:(k + 1) * 32], b1))
    return jnp.concatenate(parts, axis=0)


def _ta_mul(a, b):
    c1 = a[0] + b[0]
    c2 = a[1] + b[1] + _tp(a[0], b[0])
    c3 = _mul3(a, b)
    c4 = _mul4(a, b)
    return (c1, c2, c3, c4)


def _chen_step(carry, d):
    a1, a2, a3, a4 = carry
    e2 = _tp(d * 0.5, d)
    e3 = _tp(e2, d * (1.0 / 3.0))
    dq = d * 0.25
    c1 = a1 + d
    c2 = a2 + e2 + _tp(a1, d)
    p3 = []
    for m in range(_C):
        p3.append(a3[m * 64:(m + 1) * 64] + e3[m * 64:(m + 1) * 64]
                  + a1[m:m + 1] * e2
                  + _tp(a2[m * 8:(m + 1) * 8], d))
    p4 = []
    for k in range(2 * _C):
        m, h = k >> 1, k & 1
        p4.append(a4[k * 256:(k + 1) * 256]
                  + _tp(e3[k * 32:(k + 1) * 32], dq)
                  + a1[m:m + 1] * e3[h * 256:(h + 1) * 256]
                  + _tp(a2[k * 4:(k + 1) * 4], e2)
                  + _tp(a3[k * 32:(k + 1) * 32], d))
    return (c1, c2, jnp.concatenate(p3, axis=0), jnp.concatenate(p4, axis=0))


def _blamem_kernel(inc_ref, w1_ref, b1_ref, w2_ref, b2_ref, out_ref):
    carry0 = _exp1(inc_ref[0, 0])

    def chen_body(s, carry):
        d = inc_ref[0, s]
        return _chen_step(carry, d)

    sig = jax.lax.fori_loop(1, _L, chen_body, carry0)

    lane = jax.lax.broadcasted_iota(jnp.int32, (1, _N), 1)

    def scan_body(i, pref):
        d = jax.lax.shift_left(jnp.int32(1), i)
        maskf = (lane >= d).astype(jnp.float32)
        shifted = tuple(pltpu.roll(lv, d, 1) * maskf for lv in pref)
        return _ta_mul(shifted, pref)

    s1, s2, s3, s4 = jax.lax.fori_loop(0, _ROUNDS, scan_body, sig)

    s1h = s1 * -0.5
    s2h = s2 * -0.5
    s3h = s3 * -0.5
    s1t = s1 * (1.0 / 3.0)
    s2t = s2 * (1.0 / 3.0)
    s1q = s1 * -0.25
    p2 = _tp(s1, s1)
    p3 = _tp(s1, s2) + _tp(s2, s1)
    q3 = _tp(s1, p2)
    l1 = s1
    l2 = s2 - 0.5 * p2
    l3 = s3 - 0.5 * p3 + (1.0 / 3.0) * q3
    l4_parts = []
    for k in range(2 * _C):
        m, h = k >> 1, k & 1
        l4_parts.append(
            s4[k * 256:(k + 1) * 256]
            + s1h[m:m + 1] * s3[h * 256:(h + 1) * 256]
            + _tp(s2h[k * 4:(k + 1) * 4], s2)
            + _tp(s3h[k * 32:(k + 1) * 32], s1)
            + s1t[m:m + 1] * p3[h * 256:(h + 1) * 256]
            + _tp(s2t[k * 4:(k + 1) * 4], p2)
            + s1q[m:m + 1] * q3[h * 256:(h + 1) * 256])
    l4 = jnp.concatenate(l4_parts, axis=0)

    m1 = jnp.mean(l1, axis=1, keepdims=True)
    m2 = jnp.mean(l2, axis=1, keepdims=True)
    m3 = jnp.mean(l3, axis=1, keepdims=True)
    m4 = jnp.mean(l4, axis=1, keepdims=True)

    dn = (((0,), (0,)), ((), ()))
    h = (jax.lax.dot_general(m1, w1_ref[0:8, :], dn,
                             preferred_element_type=jnp.float32)
         + jax.lax.dot_general(m2, w1_ref[8:72, :], dn,
                               preferred_element_type=jnp.float32)
         + jax.lax.dot_general(m3, w1_ref[72:584, :], dn,
                               preferred_element_type=jnp.float32)
         + jax.lax.dot_general(m4, w1_ref[584:4680, :], dn,
                               preferred_element_type=jnp.float32)
         + b1_ref[...])
    h = jnp.maximum(h, 0.0)
    out_ref[...] = (jnp.dot(h, w2_ref[...], preferred_element_type=jnp.float32)
                    + b2_ref[...])[None]


def kernel(x, W1, b1, W2, b2):
    B, T, Cin = x.shape
    C = Cin + 1
    N = T // _L
    H = W1.shape[1]

    t = jnp.linspace(0.0, 1.0, T, dtype=x.dtype)
    path = jnp.concatenate(
        [x, jnp.broadcast_to(t[None, :, None], (B, T, 1)).astype(x.dtype)],
        axis=-1)
    inc = jnp.diff(path, axis=1, prepend=jnp.zeros((B, 1, C), x.dtype))
    inc_t = inc.reshape(B, N, _L, C).transpose(0, 2, 3, 1)

    b1_2d = b1.reshape(1, H)
    b2_2d = b2.reshape(1, 1)

    out = pl.pallas_call(
        _blamem_kernel,
        grid=(B,),
        in_specs=[
            pl.BlockSpec((1, _L, C, N), lambda b: (b, 0, 0, 0)),
            pl.BlockSpec(W1.shape, lambda b: (0, 0)),
            pl.BlockSpec((1, H), lambda b: (0, 0)),
            pl.BlockSpec(W2.shape, lambda b: (0, 0)),
            pl.BlockSpec((1, 1), lambda b: (0, 0)),
        ],
        out_specs=pl.BlockSpec((1, 1, 1), lambda b: (b, 0, 0)),
        out_shape=jax.ShapeDtypeStruct((B, 1, 1), jnp.float32),
        compiler_params=pltpu.CompilerParams(
            dimension_semantics=("arbitrary",),
            vmem_limit_bytes=56 * 1024 * 1024,
        ),
    )(inc_t, W1, b1_2d, W2, b2_2d)
    return out.reshape(B, 1)


# speedup vs baseline: 1.2633x; 1.0268x over previous
import numpy as np

import jax
import jax.numpy as jnp
from jax.experimental import pallas as pl
from jax.experimental.pallas import tpu as pltpu

_C = 8
_L = 16
_N = 128
_ROUNDS = 7


def _tp(a, b):
    A, n = a.shape
    Bd = b.shape[0]
    return (a[:, None, :] * b[None, :, :]).reshape(A * Bd, n)


def _exp1(d):
    e2 = _tp(d * 0.5, d)
    e3 = _tp(e2, d * (1.0 / 3.0))
    e4 = _tp(e3, d * 0.25)
    return (d, e2, e3, e4)


def _mul3(a, b):
    a1, a2, a3 = a[0], a[1], a[2]
    b1, b2, b3 = b[0], b[1], b[2]
    parts = []
    for m in range(_C):
        parts.append(a3[m * 64:(m + 1) * 64] + b3[m * 64:(m + 1) * 64]
                     + a1[m:m + 1] * b2
                     + _tp(a2[m * 8:(m + 1) * 8], b1))
    return jnp.concatenate(parts, axis=0)


def _mul4(a, b):
    a1, a2, a3, a4 = a
    b1, b2, b3, b4 = b
    parts = []
    for k in range(4 * _C):
        m, h = k >> 2, k & 3
        parts.append(a4[k * 128:(k + 1) * 128] + b4[k * 128:(k + 1) * 128]
                     + a1[m:m + 1] * b3[h * 128:(h + 1) * 128]
                     + _tp(a2[k * 2:(k + 1) * 2], b2)
                     + _tp(a3[k * 16:(k + 1) * 16], b1))
    return jnp.concatenate(parts, axis=0)


def _ta_mul(a, b):
    c1 = a[0] + b[0]
    c2 = a[1] + b[1] + _tp(a[0], b[0])
    c3 = _mul3(a, b)
    c4 = _mul4(a, b)
    return (c1, c2, c3, c4)


def _chen_step(carry, d):
    a1, a2, a3, a4 = carry
    e2 = _tp(d * 0.5, d)
    e3 = _tp(e2, d * (1.0 / 3.0))
    dq = d * 0.25
    c1 = a1 + d
    c2 = a2 + e2 + _tp(a1, d)
    p3 = []
    for m in range(_C):
        p3.append(a3[m * 64:(m + 1) * 64] + e3[m * 64:(m + 1) * 64]
                  + a1[m:m + 1] * e2
                  + _tp(a2[m * 8:(m + 1) * 8], d))
    p4 = []
    for k in range(4 * _C):
        m, h = k >> 2, k & 3
        p4.append(a4[k * 128:(k + 1) * 128]
                  + _tp(e3[k * 16:(k + 1) * 16], dq)
                  + a1[m:m + 1] * e3[h * 128:(h + 1) * 128]
                  + _tp(a2[k * 2:(k + 1) * 2], e2)
                  + _tp(a3[k * 16:(k + 1) * 16], d))
    return (c1, c2, jnp.concatenate(p3, axis=0), jnp.concatenate(p4, axis=0))


def _blamem_kernel(inc_ref, w1_ref, b1_ref, w2_ref, b2_ref, out_ref):
    carry0 = _exp1(inc_ref[0, 0])

    def chen_body(s, carry):
        d = inc_ref[0, s]
        return _chen_step(carry, d)

    sig = jax.lax.fori_loop(1, _L, chen_body, carry0)

    lane = jax.lax.broadcasted_iota(jnp.int32, (1, _N), 1)

    def scan_body(i, pref):
        d = jax.lax.shift_left(jnp.int32(1), i)
        maskf = (lane >= d).astype(jnp.float32)
        shifted = tuple(pltpu.roll(lv, d, 1) * maskf for lv in pref)
        return _ta_mul(shifted, pref)

    s1, s2, s3, s4 = jax.lax.fori_loop(0, _ROUNDS, scan_body, sig)

    s1h = s1 * -0.5
    s2h = s2 * -0.5
    s3h = s3 * -0.5
    s1t = s1 * (1.0 / 3.0)
    s2t = s2 * (1.0 / 3.0)
    s1q = s1 * -0.25
    p2 = _tp(s1, s1)
    p3 = _tp(s1, s2) + _tp(s2, s1)
    q3 = _tp(s1, p2)
    l1 = s1
    l2 = s2 - 0.5 * p2
    l3 = s3 - 0.5 * p3 + (1.0 / 3.0) * q3
    l4_parts = []
    for k in range(4 * _C):
        m, h = k >> 2, k & 3
        l4_parts.append(
            s4[k * 128:(k + 1) * 128]
            + s1h[m:m + 1] * s3[h * 128:(h + 1) * 128]
            + _tp(s2h[k * 2:(k + 1) * 2], s2)
            + _tp(s3h[k * 16:(k + 1) * 16], s1)
            + s1t[m:m + 1] * p3[h * 128:(h + 1) * 128]
            + _tp(s2t[k * 2:(k + 1) * 2], p2)
            + s1q[m:m + 1] * q3[h * 128:(h + 1) * 128])
    l4 = jnp.concatenate(l4_parts, axis=0)

    m1 = jnp.mean(l1, axis=1, keepdims=True)
    m2 = jnp.mean(l2, axis=1, keepdims=True)
    m3 = jnp.mean(l3, axis=1, keepdims=True)
    m4 = jnp.mean(l4, axis=1, keepdims=True)

    dn = (((0,), (0,)), ((), ()))
    h = (jax.lax.dot_general(m1, w1_ref[0:8, :], dn,
                             preferred_element_type=jnp.float32)
         + jax.lax.dot_general(m2, w1_ref[8:72, :], dn,
                               preferred_element_type=jnp.float32)
         + jax.lax.dot_general(m3, w1_ref[72:584, :], dn,
                               preferred_element_type=jnp.float32)
         + jax.lax.dot_general(m4, w1_ref[584:4680, :], dn,
                               preferred_element_type=jnp.float32)
         + b1_ref[...])
    h = jnp.maximum(h, 0.0)
    out_ref[...] = (jnp.dot(h, w2_ref[...], preferred_element_type=jnp.float32)
                    + b2_ref[...])[None]


def kernel(x, W1, b1, W2, b2):
    B, T, Cin = x.shape
    C = Cin + 1
    N = T // _L
    H = W1.shape[1]

    t = jnp.linspace(0.0, 1.0, T, dtype=x.dtype)
    path = jnp.concatenate(
        [x, jnp.broadcast_to(t[None, :, None], (B, T, 1)).astype(x.dtype)],
        axis=-1)
    inc = jnp.diff(path, axis=1, prepend=jnp.zeros((B, 1, C), x.dtype))
    inc_t = inc.reshape(B, N, _L, C).transpose(0, 2, 3, 1)

    b1_2d = b1.reshape(1, H)
    b2_2d = b2.reshape(1, 1)

    out = pl.pallas_call(
        _blamem_kernel,
        grid=(B,),
        in_specs=[
            pl.BlockSpec((1, _L, C, N), lambda b: (b, 0, 0, 0)),
            pl.BlockSpec(W1.shape, lambda b: (0, 0)),
            pl.BlockSpec((1, H), lambda b: (0, 0)),
            pl.BlockSpec(W2.shape, lambda b: (0, 0)),
            pl.BlockSpec((1, 1), lambda b: (0, 0)),
        ],
        out_specs=pl.BlockSpec((1, 1, 1), lambda b: (b, 0, 0)),
        out_shape=jax.ShapeDtypeStruct((B, 1, 1), jnp.float32),
        compiler_params=pltpu.CompilerParams(
            dimension_semantics=("arbitrary",),
            vmem_limit_bytes=56 * 1024 * 1024,
        ),
    )(inc_t, W1, b1_2d, W2, b2_2d)
    return out.reshape(B, 1)


# speedup vs baseline: 1.2648x; 1.0012x over previous
import numpy as np

import jax
import jax.numpy as jnp
from jax.experimental import pallas as pl
from jax.experimental.pallas import tpu as pltpu

_C = 8
_L = 16
_N = 128
_ROUNDS = 7


def _tp(a, b):
    A, n = a.shape
    Bd = b.shape[0]
    return (a[:, None, :] * b[None, :, :]).reshape(A * Bd, n)


def _exp1(d):
    e2 = _tp(d * 0.5, d)
    e3 = _tp(e2, d * (1.0 / 3.0))
    e4 = _tp(e3, d * 0.25)
    return (d, e2, e3, e4)


def _mul3(a, b):
    a1, a2, a3 = a[0], a[1], a[2]
    b1, b2, b3 = b[0], b[1], b[2]
    parts = []
    for m in range(_C):
        parts.append(a3[m * 64:(m + 1) * 64] + b3[m * 64:(m + 1) * 64]
                     + a1[m:m + 1] * b2
                     + _tp(a2[m * 8:(m + 1) * 8], b1))
    return jnp.concatenate(parts, axis=0)


def _mul4(a, b):
    a1, a2, a3, a4 = a
    b1, b2, b3, b4 = b
    parts = []
    for k in range(8 * _C):
        m, h = k >> 3, k & 7
        parts.append(a4[k * 64:(k + 1) * 64] + b4[k * 64:(k + 1) * 64]
                     + a1[m:m + 1] * b3[h * 64:(h + 1) * 64]
                     + a2[k:k + 1] * b2
                     + _tp(a3[k * 8:(k + 1) * 8], b1))
    return jnp.concatenate(parts, axis=0)


def _ta_mul(a, b):
    c1 = a[0] + b[0]
    c2 = a[1] + b[1] + _tp(a[0], b[0])
    c3 = _mul3(a, b)
    c4 = _mul4(a, b)
    return (c1, c2, c3, c4)


def _chen_step(carry, d):
    a1, a2, a3, a4 = carry
    e2 = _tp(d * 0.5, d)
    e3 = _tp(e2, d * (1.0 / 3.0))
    dq = d * 0.25
    c1 = a1 + d
    c2 = a2 + e2 + _tp(a1, d)
    p3 = []
    for m in range(_C):
        p3.append(a3[m * 64:(m + 1) * 64] + e3[m * 64:(m + 1) * 64]
                  + a1[m:m + 1] * e2
                  + _tp(a2[m * 8:(m + 1) * 8], d))
    p4 = []
    for k in range(8 * _C):
        m, h = k >> 3, k & 7
        p4.append(a4[k * 64:(k + 1) * 64]
                  + _tp(e3[k * 8:(k + 1) * 8], dq)
                  + a1[m:m + 1] * e3[h * 64:(h + 1) * 64]
                  + a2[k:k + 1] * e2
                  + _tp(a3[k * 8:(k + 1) * 8], d))
    return (c1, c2, jnp.concatenate(p3, axis=0), jnp.concatenate(p4, axis=0))


def _blamem_kernel(inc_ref, w1_ref, b1_ref, w2_ref, b2_ref, out_ref):
    carry0 = _exp1(inc_ref[0, 0])

    def chen_body(s, carry):
        d = inc_ref[0, s]
        return _chen_step(carry, d)

    sig = jax.lax.fori_loop(1, _L, chen_body, carry0)

    lane = jax.lax.broadcasted_iota(jnp.int32, (1, _N), 1)

    def scan_body(i, pref):
        d = jax.lax.shift_left(jnp.int32(1), i)
        maskf = (lane >= d).astype(jnp.float32)
        shifted = tuple(pltpu.roll(lv, d, 1) * maskf for lv in pref)
        return _ta_mul(shifted, pref)

    s1, s2, s3, s4 = jax.lax.fori_loop(0, _ROUNDS, scan_body, sig)

    s1h = s1 * -0.5
    s2h = s2 * -0.5
    s3h = s3 * -0.5
    s1t = s1 * (1.0 / 3.0)
    s2t = s2 * (1.0 / 3.0)
    s1q = s1 * -0.25
    p2 = _tp(s1, s1)
    p3 = _tp(s1, s2) + _tp(s2, s1)
    q3 = _tp(s1, p2)
    l1 = s1
    l2 = s2 - 0.5 * p2
    l3 = s3 - 0.5 * p3 + (1.0 / 3.0) * q3
    l4_parts = []
    for k in range(8 * _C):
        m, h = k >> 3, k & 7
        l4_parts.append(
            s4[k * 64:(k + 1) * 64]
            + s1h[m:m + 1] * s3[h * 64:(h + 1) * 64]
            + s2h[k:k + 1] * s2
            + _tp(s3h[k * 8:(k + 1) * 8], s1)
            + s1t[m:m + 1] * p3[h * 64:(h + 1) * 64]
            + s2t[k:k + 1] * p2
            + s1q[m:m + 1] * q3[h * 64:(h + 1) * 64])
    l4 = jnp.concatenate(l4_parts, axis=0)

    m1 = jnp.mean(l1, axis=1, keepdims=True)
    m2 = jnp.mean(l2, axis=1, keepdims=True)
    m3 = jnp.mean(l3, axis=1, keepdims=True)
    m4 = jnp.mean(l4, axis=1, keepdims=True)

    dn = (((0,), (0,)), ((), ()))
    h = (jax.lax.dot_general(m1, w1_ref[0:8, :], dn,
                             preferred_element_type=jnp.float32)
         + jax.lax.dot_general(m2, w1_ref[8:72, :], dn,
                               preferred_element_type=jnp.float32)
         + jax.lax.dot_general(m3, w1_ref[72:584, :], dn,
                               preferred_element_type=jnp.float32)
         + jax.lax.dot_general(m4, w1_ref[584:4680, :], dn,
                               preferred_element_type=jnp.float32)
         + b1_ref[...])
    h = jnp.maximum(h, 0.0)
    out_ref[...] = (jnp.dot(h, w2_ref[...], preferred_element_type=jnp.float32)
                    + b2_ref[...])[None]


def kernel(x, W1, b1, W2, b2):
    B, T, Cin = x.shape
    C = Cin + 1
    N = T // _L
    H = W1.shape[1]

    t = jnp.linspace(0.0, 1.0, T, dtype=x.dtype)
    path = jnp.concatenate(
        [x, jnp.broadcast_to(t[None, :, None], (B, T, 1)).astype(x.dtype)],
        axis=-1)
    inc = jnp.diff(path, axis=1, prepend=jnp.zeros((B, 1, C), x.dtype))
    inc_t = inc.reshape(B, N, _L, C).transpose(0, 2, 3, 1)

    b1_2d = b1.reshape(1, H)
    b2_2d = b2.reshape(1, 1)

    out = pl.pallas_call(
        _blamem_kernel,
        grid=(B,),
        in_specs=[
            pl.BlockSpec((1, _L, C, N), lambda b: (b, 0, 0, 0)),
            pl.BlockSpec(W1.shape, lambda b: (0, 0)),
            pl.BlockSpec((1, H), lambda b: (0, 0)),
            pl.BlockSpec(W2.shape, lambda b: (0, 0)),
            pl.BlockSpec((1, 1), lambda b: (0, 0)),
        ],
        out_specs=pl.BlockSpec((1, 1, 1), lambda b: (b, 0, 0)),
        out_shape=jax.ShapeDtypeStruct((B, 1, 1), jnp.float32),
        compiler_params=pltpu.CompilerParams(
            dimension_semantics=("arbitrary",),
            vmem_limit_bytes=56 * 1024 * 1024,
        ),
    )(inc_t, W1, b1_2d, W2, b2_2d)
    return out.reshape(B, 1)


# speedup vs baseline: 1.2680x; 1.0026x over previous
import jax
import jax.numpy as jnp
from jax.experimental import pallas as pl
from jax.experimental.pallas import tpu as pltpu

_C = 8
_L = 16
_N = 128
_ROUNDS = 7


def _tp(a, b):
    A, n = a.shape
    Bd = b.shape[0]
    return (a[:, None, :] * b[None, :, :]).reshape(A * Bd, n)


def _chen_step(carry, d):
    a1, a2, a3s, a4s = carry
    e2 = _tp(d * 0.5, d)
    dt = d * (1.0 / 3.0)
    dq = d * 0.25
    e3s = tuple(_tp(e2[m * 8:(m + 1) * 8], dt) for m in range(_C))
    c1 = a1 + d
    c2 = a2 + e2 + _tp(a1, d)
    c3s = tuple(a3s[m] + e3s[m]
                + a1[m:m + 1] * e2
                + _tp(a2[m * 8:(m + 1) * 8], d)
                for m in range(_C))
    c4s = []
    for k in range(8 * _C):
        m, h = k >> 3, k & 7
        r = slice(h * 8, h * 8 + 8)
        c4s.append(a4s[k]
                   + _tp(e3s[m][r], dq)
                   + a1[m:m + 1] * e3s[h]
                   + a2[k:k + 1] * e2
                   + _tp(a3s[m][r], d))
    return (c1, c2, c3s, tuple(c4s))


def _ta_mul(a, b):
    a1, a2, a3s, a4s = a
    b1, b2, b3s, b4s = b
    c1 = a1 + b1
    c2 = a2 + b2 + _tp(a1, b1)
    c3s = tuple(a3s[m] + b3s[m]
                + a1[m:m + 1] * b2
                + _tp(a2[m * 8:(m + 1) * 8], b1)
                for m in range(_C))
    c4s = []
    for k in range(8 * _C):
        m, h = k >> 3, k & 7
        r = slice(h * 8, h * 8 + 8)
        c4s.append(a4s[k] + b4s[k]
                   + a1[m:m + 1] * b3s[h]
                   + a2[k:k + 1] * b2
                   + _tp(a3s[m][r], b1))
    return (c1, c2, c3s, tuple(c4s))


def _blamem_kernel(inc_ref, w1_ref, b1_ref, w2_ref, b2_ref, out_ref):
    d0 = inc_ref[0, 0]
    e2_0 = _tp(d0 * 0.5, d0)
    dt0 = d0 * (1.0 / 3.0)
    dq0 = d0 * 0.25
    e3s_0 = tuple(_tp(e2_0[m * 8:(m + 1) * 8], dt0) for m in range(_C))
    e4s_0 = []
    for k in range(8 * _C):
        m, h = k >> 3, k & 7
        e4s_0.append(_tp(e3s_0[m][h * 8:h * 8 + 8], dq0))
    carry0 = (d0, e2_0, e3s_0, tuple(e4s_0))

    def chen_body(s, carry):
        return _chen_step(carry, inc_ref[0, s])

    sig = jax.lax.fori_loop(1, _L, chen_body, carry0)

    lane = jax.lax.broadcasted_iota(jnp.int32, (1, _N), 1)

    def scan_body(i, pref):
        p1, p2_, p3s, p4s = pref
        dsh = jax.lax.shift_left(jnp.int32(1), i)
        maskf = (lane >= dsh).astype(jnp.float32)

        def sh(lv):
            return pltpu.roll(lv, dsh, 1) * maskf

        shifted = (sh(p1), sh(p2_),
                   tuple(sh(v) for v in p3s),
                   tuple(sh(v) for v in p4s))
        return _ta_mul(shifted, pref)

    s1, s2, s3s, s4s = jax.lax.fori_loop(0, _ROUNDS, scan_body, sig)

    s1h = s1 * -0.5
    s2h = s2 * -0.5
    s1t = s1 * (1.0 / 3.0)
    s2t = s2 * (1.0 / 3.0)
    s1q = s1 * -0.25
    p2 = _tp(s1, s1)
    p3s = tuple(s1[m:m + 1] * s2 + _tp(s2[m * 8:(m + 1) * 8], s1)
                for m in range(_C))
    q3s = tuple(s1[m:m + 1] * p2 for m in range(_C))
    s3hs = tuple(v * -0.5 for v in s3s)
    l1 = s1
    l2 = s2 - 0.5 * p2
    l3s = tuple(s3s[m] - 0.5 * p3s[m] + (1.0 / 3.0) * q3s[m]
                for m in range(_C))
    l4s = []
    for k in range(8 * _C):
        m, h = k >> 3, k & 7
        r = slice(h * 8, h * 8 + 8)
        l4s.append(s4s[k]
                   + s1h[m:m + 1] * s3s[h]
                   + s2h[k:k + 1] * s2
                   + _tp(s3hs[m][r], s1)
                   + s1t[m:m + 1] * p3s[h]
                   + s2t[k:k + 1] * p2
                   + s1q[m:m + 1] * q3s[h])

    m12 = jnp.concatenate(
        [jnp.mean(l1, axis=1, keepdims=True),
         jnp.mean(l2, axis=1, keepdims=True)], axis=0)
    m3 = jnp.concatenate(
        [jnp.mean(v, axis=1, keepdims=True) for v in l3s], axis=0)
    m4 = jnp.concatenate(
        [jnp.mean(v, axis=1, keepdims=True) for v in l4s], axis=0)

    dn = (((0,), (0,)), ((), ()))
    h = (jax.lax.dot_general(m12, w1_ref[0:72, :], dn,
                             preferred_element_type=jnp.float32)
         + jax.lax.dot_general(m3, w1_ref[72:584, :], dn,
                               preferred_element_type=jnp.float32)
         + jax.lax.dot_general(m4, w1_ref[584:4680, :], dn,
                               preferred_element_type=jnp.float32)
         + b1_ref[...])
    h = jnp.maximum(h, 0.0)
    out_ref[...] = (jnp.dot(h, w2_ref[...], preferred_element_type=jnp.float32)
                    + b2_ref[...])[None]


def kernel(x, W1, b1, W2, b2):
    B, T, Cin = x.shape
    C = Cin + 1
    N = T // _L
    H = W1.shape[1]

    t = jnp.linspace(0.0, 1.0, T, dtype=x.dtype)
    path = jnp.concatenate(
        [x, jnp.broadcast_to(t[None, :, None], (B, T, 1)).astype(x.dtype)],
        axis=-1)
    inc = jnp.diff(path, axis=1, prepend=jnp.zeros((B, 1, C), x.dtype))
    inc_t = inc.reshape(B, N, _L, C).transpose(0, 2, 3, 1)

    b1_2d = b1.reshape(1, H)
    b2_2d = b2.reshape(1, 1)

    out = pl.pallas_call(
        _blamem_kernel,
        grid=(B,),
        in_specs=[
            pl.BlockSpec((1, _L, C, N), lambda b: (b, 0, 0, 0)),
            pl.BlockSpec(W1.shape, lambda b: (0, 0)),
            pl.BlockSpec((1, H), lambda b: (0, 0)),
            pl.BlockSpec(W2.shape, lambda b: (0, 0)),
            pl.BlockSpec((1, 1), lambda b: (0, 0)),
        ],
        out_specs=pl.BlockSpec((1, 1, 1), lambda b: (b, 0, 0)),
        out_shape=jax.ShapeDtypeStruct((B, 1, 1), jnp.float32),
        compiler_params=pltpu.CompilerParams(
            dimension_semantics=("arbitrary",),
            vmem_limit_bytes=56 * 1024 * 1024,
        ),
    )(inc_t, W1, b1_2d, W2, b2_2d)
    return out.reshape(B, 1)


# speedup vs baseline: 1.4861x; 1.1720x over previous
import jax
import jax.numpy as jnp
from jax.experimental import pallas as pl
from jax.experimental.pallas import tpu as pltpu

_C = 8
_L = 16
_N = 128
_ROUNDS = 7


def _tp(a, b):
    A, n = a.shape
    Bd = b.shape[0]
    return (a[:, None, :] * b[None, :, :]).reshape(A * Bd, n)


def _rot2(v):
    return v.reshape(_C, _C, _N).transpose(1, 0, 2).reshape(_C * _C, _N)


def _rot3(slices):
    out = []
    for j in range(_C):
        out.append(jnp.concatenate(
            [s.reshape(_C, _C, _N)[:, j, :] for s in slices], axis=0))
    return tuple(out)


def _chen_step(carry, d):
    a1, a2, a3s, a4s = carry
    dh = d * 0.5
    dt = d * (1.0 / 3.0)
    dq = d * 0.25
    e2 = _tp(dh, d)
    e2r = _tp(d, dh)
    e3s = tuple(_tp(e2[m * 8:(m + 1) * 8], dt) for m in range(_C))
    e3r = tuple(e2 * dt[j:j + 1] for j in range(_C))
    c1 = a1 + d
    c2 = a2 + e2 + _tp(a1, d)
    c3s = tuple(a3s[m] + e3s[m]
                + a1[m:m + 1] * e2
                + _tp(a2[m * 8:(m + 1) * 8], d)
                for m in range(_C))
    c4s = []
    for k in range(8 * _C):
        j, m = k >> 3, k & 7
        c4s.append(a4s[k]
                   + e3s[m] * dq[j:j + 1]
                   + a1[m:m + 1] * e3r[j]
                   + _tp(a2[m * 8:(m + 1) * 8], e2r[j * 8:(j + 1) * 8])
                   + a3s[m] * d[j:j + 1])
    return (c1, c2, c3s, tuple(c4s))


def _ta_mul(a, b):
    a1, a2, a3s, a4s = a
    b1, b2, b3s, b4s = b
    b2r = _rot2(b2)
    b3r = _rot3(b3s)
    c1 = a1 + b1
    c2 = a2 + b2 + _tp(a1, b1)
    c3s = tuple(a3s[m] + b3s[m]
                + a1[m:m + 1] * b2
                + _tp(a2[m * 8:(m + 1) * 8], b1)
                for m in range(_C))
    c4s = []
    for k in range(8 * _C):
        j, m = k >> 3, k & 7
        c4s.append(a4s[k] + b4s[k]
                   + a1[m:m + 1] * b3r[j]
                   + _tp(a2[m * 8:(m + 1) * 8], b2r[j * 8:(j + 1) * 8])
                   + a3s[m] * b1[j:j + 1])
    return (c1, c2, c3s, tuple(c4s))


def _blamem_kernel(inc_ref, w1_ref, b1_ref, w2_ref, b2_ref, out_ref):
    d0 = inc_ref[0, 0]
    dh0 = d0 * 0.5
    dt0 = d0 * (1.0 / 3.0)
    dq0 = d0 * 0.25
    e2_0 = _tp(dh0, d0)
    e3s_0 = tuple(_tp(e2_0[m * 8:(m + 1) * 8], dt0) for m in range(_C))
    e4s_0 = []
    for k in range(8 * _C):
        j, m = k >> 3, k & 7
        e4s_0.append(e3s_0[m] * dq0[j:j + 1])
    carry0 = (d0, e2_0, e3s_0, tuple(e4s_0))

    def chen_body(s, carry):
        return _chen_step(carry, inc_ref[0, s])

    sig = jax.lax.fori_loop(1, _L, chen_body, carry0)

    lane = jax.lax.broadcasted_iota(jnp.int32, (1, _N), 1)

    def scan_body(i, pref):
        p1, p2_, p3s, p4s = pref
        dsh = jax.lax.shift_left(jnp.int32(1), i)
        maskf = (lane >= dsh).astype(jnp.float32)

        def sh(lv):
            return pltpu.roll(lv, dsh, 1) * maskf

        shifted = (sh(p1), sh(p2_),
                   tuple(sh(v) for v in p3s),
                   tuple(sh(v) for v in p4s))
        return _ta_mul(shifted, pref)

    s1, s2, s3s, s4s = jax.lax.fori_loop(0, _ROUNDS, scan_body, sig)

    s1h = s1 * -0.5
    s1t = s1 * (1.0 / 3.0)
    s1q = s1 * -0.25
    p2 = _tp(s1, s1)
    s2r = _rot2(s2)
    s3r = _rot3(s3s)
    s2rh = s2r * -0.5
    p2t = p2 * (1.0 / 3.0)
    p3s = tuple(s1[m:m + 1] * s2 + _tp(s2[m * 8:(m + 1) * 8], s1)
                for m in range(_C))
    q3s = tuple(s1[m:m + 1] * p2 for m in range(_C))
    p3r = tuple(_tp(s1, s2r[j * 8:(j + 1) * 8]) + s2 * s1[j:j + 1]
                for j in range(_C))
    q3r = tuple(_tp(s1, p2[j * 8:(j + 1) * 8]) for j in range(_C))
    l1 = s1
    l2 = s2 - 0.5 * p2
    l3s = tuple(s3s[m] - 0.5 * p3s[m] + (1.0 / 3.0) * q3s[m]
                for m in range(_C))
    l4s = []
    for k in range(8 * _C):
        j, m = k >> 3, k & 7
        l4s.append(s4s[k]
                   + s1h[m:m + 1] * s3r[j]
                   + _tp(s2[m * 8:(m + 1) * 8], s2rh[j * 8:(j + 1) * 8])
                   + s3s[m] * s1h[j:j + 1]
                   + s1t[m:m + 1] * p3r[j]
                   + _tp(s2[m * 8:(m + 1) * 8], p2t[j * 8:(j + 1) * 8])
                   + s1q[m:m + 1] * q3r[j])
    m12 = jnp.concatenate(
        [jnp.mean(l1, axis=1, keepdims=True),
         jnp.mean(l2, axis=1, keepdims=True)], axis=0)
    m3 = jnp.concatenate(
        [jnp.mean(v, axis=1, keepdims=True) for v in l3s], axis=0)
    m4 = jnp.concatenate(
        [jnp.mean(v, axis=1, keepdims=True) for v in l4s], axis=0)

    dn = (((0,), (0,)), ((), ()))
    h = (jax.lax.dot_general(m12, w1_ref[0:72, :], dn,
                             preferred_element_type=jnp.float32)
         + jax.lax.dot_general(m3, w1_ref[72:584, :], dn,
                               preferred_element_type=jnp.float32)
         + jax.lax.dot_general(m4, w1_ref[584:4680, :], dn,
                               preferred_element_type=jnp.float32)
         + b1_ref[...])
    h = jnp.maximum(h, 0.0)
    out_ref[...] = (jnp.dot(h, w2_ref[...], preferred_element_type=jnp.float32)
                    + b2_ref[...])[None]


def kernel(x, W1, b1, W2, b2):
    B, T, Cin = x.shape
    C = Cin + 1
    N = T // _L
    H = W1.shape[1]

    t = jnp.linspace(0.0, 1.0, T, dtype=x.dtype)
    path = jnp.concatenate(
        [x, jnp.broadcast_to(t[None, :, None], (B, T, 1)).astype(x.dtype)],
        axis=-1)
    inc = jnp.diff(path, axis=1, prepend=jnp.zeros((B, 1, C), x.dtype))
    inc_t = inc.reshape(B, N, _L, C).transpose(0, 2, 3, 1)

    W1r = jnp.concatenate(
        [W1[:584],
         W1[584:].reshape(C, C, C, C, H).transpose(3, 0, 1, 2, 4)
                 .reshape(C ** 4, H)], axis=0)

    b1_2d = b1.reshape(1, H)
    b2_2d = b2.reshape(1, 1)

    out = pl.pallas_call(
        _blamem_kernel,
        grid=(B,),
        in_specs=[
            pl.BlockSpec((1, _L, C, N), lambda b: (b, 0, 0, 0)),
            pl.BlockSpec(W1.shape, lambda b: (0, 0)),
            pl.BlockSpec((1, H), lambda b: (0, 0)),
            pl.BlockSpec(W2.shape, lambda b: (0, 0)),
            pl.BlockSpec((1, 1), lambda b: (0, 0)),
        ],
        out_specs=pl.BlockSpec((1, 1, 1), lambda b: (b, 0, 0)),
        out_shape=jax.ShapeDtypeStruct((B, 1, 1), jnp.float32),
        compiler_params=pltpu.CompilerParams(
            dimension_semantics=("arbitrary",),
            vmem_limit_bytes=56 * 1024 * 1024,
        ),
    )(inc_t, W1r, b1_2d, W2, b2_2d)
    return out.reshape(B, 1)


# speedup vs baseline: 1.6802x; 1.1306x over previous
import jax
import jax.numpy as jnp
from jax.experimental import pallas as pl
from jax.experimental.pallas import tpu as pltpu

_C = 8
_L = 16
_N = 128
_ROUNDS = 7


def _tp(a, b):
    A, n = a.shape
    Bd = b.shape[0]
    return (a[:, None, :] * b[None, :, :]).reshape(A * Bd, n)


def _rot2(v):
    return v.reshape(_C, _C, _N).transpose(1, 0, 2).reshape(_C * _C, _N)


def _rot3(slices):
    out = []
    for j in range(_C):
        out.append(jnp.concatenate(
            [s.reshape(_C, _C, _N)[:, j, :] for s in slices], axis=0))
    return tuple(out)


def _chen_step(carry, d):
    a1, a2, a3s, a4s = carry
    dh = d * 0.5
    dt = d * (1.0 / 3.0)
    dq = d * 0.25
    e2 = _tp(dh, d)
    e2r = _tp(d, dh)
    e3s = tuple(_tp(e2[m * 8:(m + 1) * 8], dt) for m in range(_C))
    e3r = tuple(e2 * dt[j:j + 1] for j in range(_C))
    c1 = a1 + d
    c2 = a2 + e2 + _tp(a1, d)
    c3s = tuple(a3s[m] + e3s[m]
                + a1[m:m + 1] * e2
                + _tp(a2[m * 8:(m + 1) * 8], d)
                for m in range(_C))
    c4s = []
    for k in range(8 * _C):
        j, m = k >> 3, k & 7
        c4s.append(a4s[k]
                   + e3s[m] * dq[j:j + 1]
                   + a1[m:m + 1] * e3r[j]
                   + _tp(a2[m * 8:(m + 1) * 8], e2r[j * 8:(j + 1) * 8])
                   + a3s[m] * d[j:j + 1])
    return (c1, c2, c3s, tuple(c4s))


def _ta_mul(a, b):
    a1, a2, a3s, a4s = a
    b1, b2, b3s, b4s = b
    b2r = _rot2(b2)
    b3r = _rot3(b3s)
    c1 = a1 + b1
    c2 = a2 + b2 + _tp(a1, b1)
    c3s = tuple(a3s[m] + b3s[m]
                + a1[m:m + 1] * b2
                + _tp(a2[m * 8:(m + 1) * 8], b1)
                for m in range(_C))
    c4s = []
    for k in range(8 * _C):
        j, m = k >> 3, k & 7
        c4s.append(a4s[k] + b4s[k]
                   + a1[m:m + 1] * b3r[j]
                   + _tp(a2[m * 8:(m + 1) * 8], b2r[j * 8:(j + 1) * 8])
                   + a3s[m] * b1[j:j + 1])
    return (c1, c2, c3s, tuple(c4s))


def _blamem_kernel(inc_ref, w1_ref, b1_ref, w2_ref, b2_ref, out_ref):
    d0 = inc_ref[0, 0]
    dh0 = d0 * 0.5
    dt0 = d0 * (1.0 / 3.0)
    dq0 = d0 * 0.25
    e2_0 = _tp(dh0, d0)
    e3s_0 = tuple(_tp(e2_0[m * 8:(m + 1) * 8], dt0) for m in range(_C))
    e4s_0 = []
    for k in range(8 * _C):
        j, m = k >> 3, k & 7
        e4s_0.append(e3s_0[m] * dq0[j:j + 1])
    carry0 = (d0, e2_0, e3s_0, tuple(e4s_0))

    def chen_body(s, carry):
        return _chen_step(carry, inc_ref[0, s])

    sig = jax.lax.fori_loop(1, _L, chen_body, carry0, unroll=3)

    lane = jax.lax.broadcasted_iota(jnp.int32, (1, _N), 1)

    def scan_body(i, pref):
        p1, p2_, p3s, p4s = pref
        dsh = jax.lax.shift_left(jnp.int32(1), i)
        maskf = (lane >= dsh).astype(jnp.float32)

        def sh(lv):
            return pltpu.roll(lv, dsh, 1) * maskf

        shifted = (sh(p1), sh(p2_),
                   tuple(sh(v) for v in p3s),
                   tuple(sh(v) for v in p4s))
        return _ta_mul(shifted, pref)

    s1, s2, s3s, s4s = jax.lax.fori_loop(0, _ROUNDS, scan_body, sig)

    s1h = s1 * -0.5
    s1t = s1 * (1.0 / 3.0)
    s1q = s1 * -0.25
    p2 = _tp(s1, s1)
    s2r = _rot2(s2)
    s3r = _rot3(s3s)
    s2rh = s2r * -0.5
    p2t = p2 * (1.0 / 3.0)
    p3s = tuple(s1[m:m + 1] * s2 + _tp(s2[m * 8:(m + 1) * 8], s1)
                for m in range(_C))
    q3s = tuple(s1[m:m + 1] * p2 for m in range(_C))
    p3r = tuple(_tp(s1, s2r[j * 8:(j + 1) * 8]) + s2 * s1[j:j + 1]
                for j in range(_C))
    q3r = tuple(_tp(s1, p2[j * 8:(j + 1) * 8]) for j in range(_C))
    l1 = s1
    l2 = s2 - 0.5 * p2
    l3s = tuple(s3s[m] - 0.5 * p3s[m] + (1.0 / 3.0) * q3s[m]
                for m in range(_C))
    l4s = []
    for k in range(8 * _C):
        j, m = k >> 3, k & 7
        l4s.append(s4s[k]
                   + s1h[m:m + 1] * s3r[j]
                   + _tp(s2[m * 8:(m + 1) * 8], s2rh[j * 8:(j + 1) * 8])
                   + s3s[m] * s1h[j:j + 1]
                   + s1t[m:m + 1] * p3r[j]
                   + _tp(s2[m * 8:(m + 1) * 8], p2t[j * 8:(j + 1) * 8])
                   + s1q[m:m + 1] * q3r[j])
    m12 = jnp.concatenate(
        [jnp.mean(l1, axis=1, keepdims=True),
         jnp.mean(l2, axis=1, keepdims=True)], axis=0)
    m3 = jnp.concatenate(
        [jnp.mean(v, axis=1, keepdims=True) for v in l3s], axis=0)
    m4 = jnp.concatenate(
        [jnp.mean(v, axis=1, keepdims=True) for v in l4s], axis=0)

    dn = (((0,), (0,)), ((), ()))
    h = (jax.lax.dot_general(m12, w1_ref[0:72, :], dn,
                             preferred_element_type=jnp.float32)
         + jax.lax.dot_general(m3, w1_ref[72:584, :], dn,
                               preferred_element_type=jnp.float32)
         + jax.lax.dot_general(m4, w1_ref[584:4680, :], dn,
                               preferred_element_type=jnp.float32)
         + b1_ref[...])
    h = jnp.maximum(h, 0.0)
    out_ref[...] = (jnp.dot(h, w2_ref[...], preferred_element_type=jnp.float32)
                    + b2_ref[...])[None]


def kernel(x, W1, b1, W2, b2):
    B, T, Cin = x.shape
    C = Cin + 1
    N = T // _L
    H = W1.shape[1]

    t = jnp.linspace(0.0, 1.0, T, dtype=x.dtype)
    path = jnp.concatenate(
        [x, jnp.broadcast_to(t[None, :, None], (B, T, 1)).astype(x.dtype)],
        axis=-1)
    inc = jnp.diff(path, axis=1, prepend=jnp.zeros((B, 1, C), x.dtype))
    inc_t = inc.reshape(B, N, _L, C).transpose(0, 2, 3, 1)

    W1r = jnp.concatenate(
        [W1[:584],
         W1[584:].reshape(C, C, C, C, H).transpose(3, 0, 1, 2, 4)
                 .reshape(C ** 4, H)], axis=0)

    b1_2d = b1.reshape(1, H)
    b2_2d = b2.reshape(1, 1)

    out = pl.pallas_call(
        _blamem_kernel,
        grid=(B,),
        in_specs=[
            pl.BlockSpec((1, _L, C, N), lambda b: (b, 0, 0, 0)),
            pl.BlockSpec(W1.shape, lambda b: (0, 0)),
            pl.BlockSpec((1, H), lambda b: (0, 0)),
            pl.BlockSpec(W2.shape, lambda b: (0, 0)),
            pl.BlockSpec((1, 1), lambda b: (0, 0)),
        ],
        out_specs=pl.BlockSpec((1, 1, 1), lambda b: (b, 0, 0)),
        out_shape=jax.ShapeDtypeStruct((B, 1, 1), jnp.float32),
        compiler_params=pltpu.CompilerParams(
            dimension_semantics=("arbitrary",),
            vmem_limit_bytes=56 * 1024 * 1024,
        ),
    )(inc_t, W1r, b1_2d, W2, b2_2d)
    return out.reshape(B, 1)


# speedup vs baseline: 1.8335x; 1.0912x over previous
import jax
import jax.numpy as jnp
from jax.experimental import pallas as pl
from jax.experimental.pallas import tpu as pltpu

_C = 8
_L = 16
_N = 128
_ROUNDS = 7


def _tp(a, b):
    A, n = a.shape
    Bd = b.shape[0]
    return (a[:, None, :] * b[None, :, :]).reshape(A * Bd, n)


def _rot2(v):
    return v.reshape(_C, _C, _N).transpose(1, 0, 2).reshape(_C * _C, _N)


def _rot3(slices):
    out = []
    for j in range(_C):
        out.append(jnp.concatenate(
            [s.reshape(_C, _C, _N)[:, j, :] for s in slices], axis=0))
    return tuple(out)


def _chen_step(carry, d):
    a1, a2, a3s, a4s = carry
    dh = d * 0.5
    dt = d * (1.0 / 3.0)
    dq = d * 0.25
    e2 = _tp(dh, d)
    e2r = _tp(d, dh)
    e3s = tuple(_tp(e2[m * 8:(m + 1) * 8], dt) for m in range(_C))
    e3r = tuple(e2 * dt[j:j + 1] for j in range(_C))
    c1 = a1 + d
    c2 = a2 + e2 + _tp(a1, d)
    c3s = tuple(a3s[m] + e3s[m]
                + a1[m:m + 1] * e2
                + _tp(a2[m * 8:(m + 1) * 8], d)
                for m in range(_C))
    c4s = []
    for k in range(8 * _C):
        j, m = k >> 3, k & 7
        c4s.append(a4s[k]
                   + e3s[m] * dq[j:j + 1]
                   + a1[m:m + 1] * e3r[j]
                   + _tp(a2[m * 8:(m + 1) * 8], e2r[j * 8:(j + 1) * 8])
                   + a3s[m] * d[j:j + 1])
    return (c1, c2, c3s, tuple(c4s))


def _ta_mul(a, b):
    a1, a2, a3s, a4s = a
    b1, b2, b3s, b4s = b
    b2r = _rot2(b2)
    b3r = _rot3(b3s)
    c1 = a1 + b1
    c2 = a2 + b2 + _tp(a1, b1)
    c3s = tuple(a3s[m] + b3s[m]
                + a1[m:m + 1] * b2
                + _tp(a2[m * 8:(m + 1) * 8], b1)
                for m in range(_C))
    c4s = []
    for k in range(8 * _C):
        j, m = k >> 3, k & 7
        c4s.append(a4s[k] + b4s[k]
                   + a1[m:m + 1] * b3r[j]
                   + _tp(a2[m * 8:(m + 1) * 8], b2r[j * 8:(j + 1) * 8])
                   + a3s[m] * b1[j:j + 1])
    return (c1, c2, c3s, tuple(c4s))


def _blamem_kernel(inc_ref, w1_ref, b1_ref, w2_ref, b2_ref, out_ref):
    d0 = inc_ref[0, 0]
    dh0 = d0 * 0.5
    dt0 = d0 * (1.0 / 3.0)
    dq0 = d0 * 0.25
    e2_0 = _tp(dh0, d0)
    e3s_0 = tuple(_tp(e2_0[m * 8:(m + 1) * 8], dt0) for m in range(_C))
    e4s_0 = []
    for k in range(8 * _C):
        j, m = k >> 3, k & 7
        e4s_0.append(e3s_0[m] * dq0[j:j + 1])
    carry0 = (d0, e2_0, e3s_0, tuple(e4s_0))

    def chen_body(s, carry):
        return _chen_step(carry, inc_ref[0, s])

    sig = jax.lax.fori_loop(1, _L, chen_body, carry0, unroll=5)

    lane = jax.lax.broadcasted_iota(jnp.int32, (1, _N), 1)

    def scan_body(i, pref):
        p1, p2_, p3s, p4s = pref
        dsh = jax.lax.shift_left(jnp.int32(1), i)
        maskf = (lane >= dsh).astype(jnp.float32)

        def sh(lv):
            return pltpu.roll(lv, dsh, 1) * maskf

        shifted = (sh(p1), sh(p2_),
                   tuple(sh(v) for v in p3s),
                   tuple(sh(v) for v in p4s))
        return _ta_mul(shifted, pref)

    s1, s2, s3s, s4s = jax.lax.fori_loop(0, _ROUNDS, scan_body, sig, unroll=3)

    s1h = s1 * -0.5
    s1t = s1 * (1.0 / 3.0)
    s1q = s1 * -0.25
    p2 = _tp(s1, s1)
    s2r = _rot2(s2)
    s3r = _rot3(s3s)
    s2rh = s2r * -0.5
    p2t = p2 * (1.0 / 3.0)
    p3s = tuple(s1[m:m + 1] * s2 + _tp(s2[m * 8:(m + 1) * 8], s1)
                for m in range(_C))
    q3s = tuple(s1[m:m + 1] * p2 for m in range(_C))
    p3r = tuple(_tp(s1, s2r[j * 8:(j + 1) * 8]) + s2 * s1[j:j + 1]
                for j in range(_C))
    q3r = tuple(_tp(s1, p2[j * 8:(j + 1) * 8]) for j in range(_C))
    l1 = s1
    l2 = s2 - 0.5 * p2
    l3s = tuple(s3s[m] - 0.5 * p3s[m] + (1.0 / 3.0) * q3s[m]
                for m in range(_C))
    l4s = []
    for k in range(8 * _C):
        j, m = k >> 3, k & 7
        l4s.append(s4s[k]
                   + s1h[m:m + 1] * s3r[j]
                   + _tp(s2[m * 8:(m + 1) * 8], s2rh[j * 8:(j + 1) * 8])
                   + s3s[m] * s1h[j:j + 1]
                   + s1t[m:m + 1] * p3r[j]
                   + _tp(s2[m * 8:(m + 1) * 8], p2t[j * 8:(j + 1) * 8])
                   + s1q[m:m + 1] * q3r[j])
    m12 = jnp.concatenate(
        [jnp.mean(l1, axis=1, keepdims=True),
         jnp.mean(l2, axis=1, keepdims=True)], axis=0)
    m3 = jnp.concatenate(
        [jnp.mean(v, axis=1, keepdims=True) for v in l3s], axis=0)
    m4 = jnp.concatenate(
        [jnp.mean(v, axis=1, keepdims=True) for v in l4s], axis=0)

    dn = (((0,), (0,)), ((), ()))
    h = (jax.lax.dot_general(m12, w1_ref[0:72, :], dn,
                             preferred_element_type=jnp.float32)
         + jax.lax.dot_general(m3, w1_ref[72:584, :], dn,
                               preferred_element_type=jnp.float32)
         + jax.lax.dot_general(m4, w1_ref[584:4680, :], dn,
                               preferred_element_type=jnp.float32)
         + b1_ref[...])
    h = jnp.maximum(h, 0.0)
    out_ref[...] = (jnp.dot(h, w2_ref[...], preferred_element_type=jnp.float32)
                    + b2_ref[...])[None]


def kernel(x, W1, b1, W2, b2):
    B, T, Cin = x.shape
    C = Cin + 1
    N = T // _L
    H = W1.shape[1]

    t = jnp.linspace(0.0, 1.0, T, dtype=x.dtype)
    path = jnp.concatenate(
        [x, jnp.broadcast_to(t[None, :, None], (B, T, 1)).astype(x.dtype)],
        axis=-1)
    inc = jnp.diff(path, axis=1, prepend=jnp.zeros((B, 1, C), x.dtype))
    inc_t = inc.reshape(B, N, _L, C).transpose(0, 2, 3, 1)

    W1r = jnp.concatenate(
        [W1[:584],
         W1[584:].reshape(C, C, C, C, H).transpose(3, 0, 1, 2, 4)
                 .reshape(C ** 4, H)], axis=0)

    b1_2d = b1.reshape(1, H)
    b2_2d = b2.reshape(1, 1)

    out = pl.pallas_call(
        _blamem_kernel,
        grid=(B,),
        in_specs=[
            pl.BlockSpec((1, _L, C, N), lambda b: (b, 0, 0, 0)),
            pl.BlockSpec(W1.shape, lambda b: (0, 0)),
            pl.BlockSpec((1, H), lambda b: (0, 0)),
            pl.BlockSpec(W2.shape, lambda b: (0, 0)),
            pl.BlockSpec((1, 1), lambda b: (0, 0)),
        ],
        out_specs=pl.BlockSpec((1, 1, 1), lambda b: (b, 0, 0)),
        out_shape=jax.ShapeDtypeStruct((B, 1, 1), jnp.float32),
        compiler_params=pltpu.CompilerParams(
            dimension_semantics=("arbitrary",),
            vmem_limit_bytes=56 * 1024 * 1024,
        ),
    )(inc_t, W1r, b1_2d, W2, b2_2d)
    return out.reshape(B, 1)


# speedup vs baseline: 2.0246x; 1.1042x over previous
import jax
import jax.numpy as jnp
from jax.experimental import pallas as pl
from jax.experimental.pallas import tpu as pltpu

_C = 8
_L = 16
_N = 128
_ROUNDS = 7


def _tp(a, b):
    A, n = a.shape
    Bd = b.shape[0]
    return (a[:, None, :] * b[None, :, :]).reshape(A * Bd, n)


def _rot2(v):
    return v.reshape(_C, _C, _N).transpose(1, 0, 2).reshape(_C * _C, _N)


def _rot3(slices):
    out = []
    for j in range(_C):
        out.append(jnp.concatenate(
            [s.reshape(_C, _C, _N)[:, j, :] for s in slices], axis=0))
    return tuple(out)


def _chen_step(carry, d):
    a1, a2, a3s, a4s = carry
    dh = d * 0.5
    dt = d * (1.0 / 3.0)
    dq = d * 0.25
    e2 = _tp(dh, d)
    e2r = _tp(d, dh)
    e3s = tuple(_tp(e2[m * 8:(m + 1) * 8], dt) for m in range(_C))
    e3r = tuple(e2 * dt[j:j + 1] for j in range(_C))
    c1 = a1 + d
    c2 = a2 + e2 + _tp(a1, d)
    c3s = tuple(a3s[m] + e3s[m]
                + a1[m:m + 1] * e2
                + _tp(a2[m * 8:(m + 1) * 8], d)
                for m in range(_C))
    c4s = []
    for k in range(8 * _C):
        j, m = k >> 3, k & 7
        c4s.append(a4s[k]
                   + e3s[m] * dq[j:j + 1]
                   + a1[m:m + 1] * e3r[j]
                   + _tp(a2[m * 8:(m + 1) * 8], e2r[j * 8:(j + 1) * 8])
                   + a3s[m] * d[j:j + 1])
    return (c1, c2, c3s, tuple(c4s))


def _ta_mul(a, b):
    a1, a2, a3s, a4s = a
    b1, b2, b3s, b4s = b
    b2r = _rot2(b2)
    b3r = _rot3(b3s)
    c1 = a1 + b1
    c2 = a2 + b2 + _tp(a1, b1)
    c3s = tuple(a3s[m] + b3s[m]
                + a1[m:m + 1] * b2
                + _tp(a2[m * 8:(m + 1) * 8], b1)
                for m in range(_C))
    c4s = []
    for k in range(8 * _C):
        j, m = k >> 3, k & 7
        c4s.append(a4s[k] + b4s[k]
                   + a1[m:m + 1] * b3r[j]
                   + _tp(a2[m * 8:(m + 1) * 8], b2r[j * 8:(j + 1) * 8])
                   + a3s[m] * b1[j:j + 1])
    return (c1, c2, c3s, tuple(c4s))


def _blamem_kernel(inc_ref, w1_ref, b1_ref, w2_ref, b2_ref, out_ref):
    d0 = inc_ref[0, 0]
    dh0 = d0 * 0.5
    dt0 = d0 * (1.0 / 3.0)
    dq0 = d0 * 0.25
    e2_0 = _tp(dh0, d0)
    e3s_0 = tuple(_tp(e2_0[m * 8:(m + 1) * 8], dt0) for m in range(_C))
    e4s_0 = []
    for k in range(8 * _C):
        j, m = k >> 3, k & 7
        e4s_0.append(e3s_0[m] * dq0[j:j + 1])
    carry0 = (d0, e2_0, e3s_0, tuple(e4s_0))

    def chen_body(s, carry):
        return _chen_step(carry, inc_ref[0, s])

    sig = jax.lax.fori_loop(1, _L, chen_body, carry0, unroll=15)

    lane = jax.lax.broadcasted_iota(jnp.int32, (1, _N), 1)

    def scan_body(i, pref):
        p1, p2_, p3s, p4s = pref
        dsh = jax.lax.shift_left(jnp.int32(1), i)
        maskf = (lane >= dsh).astype(jnp.float32)

        def sh(lv):
            return pltpu.roll(lv, dsh, 1) * maskf

        shifted = (sh(p1), sh(p2_),
                   tuple(sh(v) for v in p3s),
                   tuple(sh(v) for v in p4s))
        return _ta_mul(shifted, pref)

    s1, s2, s3s, s4s = jax.lax.fori_loop(0, _ROUNDS, scan_body, sig, unroll=7)

    s1h = s1 * -0.5
    s1t = s1 * (1.0 / 3.0)
    s1q = s1 * -0.25
    p2 = _tp(s1, s1)
    s2r = _rot2(s2)
    s3r = _rot3(s3s)
    s2rh = s2r * -0.5
    p2t = p2 * (1.0 / 3.0)
    p3s = tuple(s1[m:m + 1] * s2 + _tp(s2[m * 8:(m + 1) * 8], s1)
                for m in range(_C))
    q3s = tuple(s1[m:m + 1] * p2 for m in range(_C))
    p3r = tuple(_tp(s1, s2r[j * 8:(j + 1) * 8]) + s2 * s1[j:j + 1]
                for j in range(_C))
    q3r = tuple(_tp(s1, p2[j * 8:(j + 1) * 8]) for j in range(_C))
    l1 = s1
    l2 = s2 - 0.5 * p2
    l3s = tuple(s3s[m] - 0.5 * p3s[m] + (1.0 / 3.0) * q3s[m]
                for m in range(_C))
    l4s = []
    for k in range(8 * _C):
        j, m = k >> 3, k & 7
        l4s.append(s4s[k]
                   + s1h[m:m + 1] * s3r[j]
                   + _tp(s2[m * 8:(m + 1) * 8], s2rh[j * 8:(j + 1) * 8])
                   + s3s[m] * s1h[j:j + 1]
                   + s1t[m:m + 1] * p3r[j]
                   + _tp(s2[m * 8:(m + 1) * 8], p2t[j * 8:(j + 1) * 8])
                   + s1q[m:m + 1] * q3r[j])
    m12 = jnp.concatenate(
        [jnp.mean(l1, axis=1, keepdims=True),
         jnp.mean(l2, axis=1, keepdims=True)], axis=0)
    m3 = jnp.concatenate(
        [jnp.mean(v, axis=1, keepdims=True) for v in l3s], axis=0)
    m4 = jnp.concatenate(
        [jnp.mean(v, axis=1, keepdims=True) for v in l4s], axis=0)

    dn = (((0,), (0,)), ((), ()))
    h = (jax.lax.dot_general(m12, w1_ref[0:72, :], dn,
                             preferred_element_type=jnp.float32)
         + jax.lax.dot_general(m3, w1_ref[72:584, :], dn,
                               preferred_element_type=jnp.float32)
         + jax.lax.dot_general(m4, w1_ref[584:4680, :], dn,
                               preferred_element_type=jnp.float32)
         + b1_ref[...])
    h = jnp.maximum(h, 0.0)
    out_ref[...] = (jnp.dot(h, w2_ref[...], preferred_element_type=jnp.float32)
                    + b2_ref[...])[None]


def kernel(x, W1, b1, W2, b2):
    B, T, Cin = x.shape
    C = Cin + 1
    N = T // _L
    H = W1.shape[1]

    t = jnp.linspace(0.0, 1.0, T, dtype=x.dtype)
    path = jnp.concatenate(
        [x, jnp.broadcast_to(t[None, :, None], (B, T, 1)).astype(x.dtype)],
        axis=-1)
    inc = jnp.diff(path, axis=1, prepend=jnp.zeros((B, 1, C), x.dtype))
    inc_t = inc.reshape(B, N, _L, C).transpose(0, 2, 3, 1)

    W1r = jnp.concatenate(
        [W1[:584],
         W1[584:].reshape(C, C, C, C, H).transpose(3, 0, 1, 2, 4)
                 .reshape(C ** 4, H)], axis=0)

    b1_2d = b1.reshape(1, H)
    b2_2d = b2.reshape(1, 1)

    out = pl.pallas_call(
        _blamem_kernel,
        grid=(B,),
        in_specs=[
            pl.BlockSpec((1, _L, C, N), lambda b: (b, 0, 0, 0)),
            pl.BlockSpec(W1.shape, lambda b: (0, 0)),
            pl.BlockSpec((1, H), lambda b: (0, 0)),
            pl.BlockSpec(W2.shape, lambda b: (0, 0)),
            pl.BlockSpec((1, 1), lambda b: (0, 0)),
        ],
        out_specs=pl.BlockSpec((1, 1, 1), lambda b: (b, 0, 0)),
        out_shape=jax.ShapeDtypeStruct((B, 1, 1), jnp.float32),
        compiler_params=pltpu.CompilerParams(
            dimension_semantics=("arbitrary",),
            vmem_limit_bytes=56 * 1024 * 1024,
        ),
    )(inc_t, W1r, b1_2d, W2, b2_2d)
    return out.reshape(B, 1)


# speedup vs baseline: 2.0251x; 1.0003x over previous
import jax
import jax.numpy as jnp
from jax.experimental import pallas as pl
from jax.experimental.pallas import tpu as pltpu

_C = 8
_L = 16
_N = 128
_ROUNDS = 7


def _tp(a, b):
    A, n = a.shape
    Bd = b.shape[0]
    return (a[:, None, :] * b[None, :, :]).reshape(A * Bd, n)


def _rot2(v):
    return v.reshape(_C, _C, _N).transpose(1, 0, 2).reshape(_C * _C, _N)


def _rot3(slices):
    out = []
    for j in range(_C):
        out.append(jnp.concatenate(
            [s.reshape(_C, _C, _N)[:, j, :] for s in slices], axis=0))
    return tuple(out)


def _chen_step(carry, d):
    a1, a2, a3s, a4s = carry
    dh = d * 0.5
    dt = d * (1.0 / 3.0)
    dq = d * 0.25
    e2 = _tp(dh, d)
    e2r = _tp(d, dh)
    e3s = tuple(_tp(e2[m * 8:(m + 1) * 8], dt) for m in range(_C))
    e3r = tuple(e2 * dt[j:j + 1] for j in range(_C))
    c1 = a1 + d
    c2 = a2 + e2 + _tp(a1, d)
    c3s = tuple(a3s[m] + e3s[m]
                + a1[m:m + 1] * e2
                + _tp(a2[m * 8:(m + 1) * 8], d)
                for m in range(_C))
    c4s = []
    for k in range(8 * _C):
        j, m = k >> 3, k & 7
        c4s.append(a4s[k]
                   + e3s[m] * dq[j:j + 1]
                   + a1[m:m + 1] * e3r[j]
                   + _tp(a2[m * 8:(m + 1) * 8], e2r[j * 8:(j + 1) * 8])
                   + a3s[m] * d[j:j + 1])
    return (c1, c2, c3s, tuple(c4s))


def _ta_mul(a, b):
    a1, a2, a3s, a4s = a
    b1, b2, b3s, b4s = b
    b2r = _rot2(b2)
    b3r = _rot3(b3s)
    c1 = a1 + b1
    c2 = a2 + b2 + _tp(a1, b1)
    c3s = tuple(a3s[m] + b3s[m]
                + a1[m:m + 1] * b2
                + _tp(a2[m * 8:(m + 1) * 8], b1)
                for m in range(_C))
    c4s = []
    for k in range(8 * _C):
        j, m = k >> 3, k & 7
        c4s.append(a4s[k] + b4s[k]
                   + a1[m:m + 1] * b3r[j]
                   + _tp(a2[m * 8:(m + 1) * 8], b2r[j * 8:(j + 1) * 8])
                   + a3s[m] * b1[j:j + 1])
    return (c1, c2, c3s, tuple(c4s))


def _blamem_kernel(inc_ref, w1_ref, b1_ref, w2_ref, b2_ref, out_ref):
    d0 = inc_ref[0, 0]
    dh0 = d0 * 0.5
    dt0 = d0 * (1.0 / 3.0)
    dq0 = d0 * 0.25
    e2_0 = _tp(dh0, d0)
    e3s_0 = tuple(_tp(e2_0[m * 8:(m + 1) * 8], dt0) for m in range(_C))
    e4s_0 = []
    for k in range(8 * _C):
        j, m = k >> 3, k & 7
        e4s_0.append(e3s_0[m] * dq0[j:j + 1])
    carry0 = (d0, e2_0, e3s_0, tuple(e4s_0))

    carry = carry0
    for s in range(1, _L):
        carry = _chen_step(carry, inc_ref[0, s])
    sig = carry

    lane = jax.lax.broadcasted_iota(jnp.int32, (1, _N), 1)

    pref = sig
    for i in range(_ROUNDS):
        p1, p2_, p3s, p4s = pref
        dsh = 1 << i
        maskf = (lane >= dsh).astype(jnp.float32)

        def sh(lv, dsh=dsh, maskf=maskf):
            return pltpu.roll(lv, dsh, 1) * maskf

        shifted = (sh(p1), sh(p2_),
                   tuple(sh(v) for v in p3s),
                   tuple(sh(v) for v in p4s))
        pref = _ta_mul(shifted, pref)

    s1, s2, s3s, s4s = pref

    s1h = s1 * -0.5
    s1t = s1 * (1.0 / 3.0)
    s1q = s1 * -0.25
    p2 = _tp(s1, s1)
    s2r = _rot2(s2)
    s3r = _rot3(s3s)
    s2rh = s2r * -0.5
    p2t = p2 * (1.0 / 3.0)
    p3s = tuple(s1[m:m + 1] * s2 + _tp(s2[m * 8:(m + 1) * 8], s1)
                for m in range(_C))
    q3s = tuple(s1[m:m + 1] * p2 for m in range(_C))
    p3r = tuple(_tp(s1, s2r[j * 8:(j + 1) * 8]) + s2 * s1[j:j + 1]
                for j in range(_C))
    q3r = tuple(_tp(s1, p2[j * 8:(j + 1) * 8]) for j in range(_C))
    l1 = s1
    l2 = s2 - 0.5 * p2
    l3s = tuple(s3s[m] - 0.5 * p3s[m] + (1.0 / 3.0) * q3s[m]
                for m in range(_C))
    l4s = []
    for k in range(8 * _C):
        j, m = k >> 3, k & 7
        l4s.append(s4s[k]
                   + s1h[m:m + 1] * s3r[j]
                   + _tp(s2[m * 8:(m + 1) * 8], s2rh[j * 8:(j + 1) * 8])
                   + s3s[m] * s1h[j:j + 1]
                   + s1t[m:m + 1] * p3r[j]
                   + _tp(s2[m * 8:(m + 1) * 8], p2t[j * 8:(j + 1) * 8])
                   + s1q[m:m + 1] * q3r[j])
    m12 = jnp.concatenate(
        [jnp.mean(l1, axis=1, keepdims=True),
         jnp.mean(l2, axis=1, keepdims=True)], axis=0)
    m3 = jnp.concatenate(
        [jnp.mean(v, axis=1, keepdims=True) for v in l3s], axis=0)
    m4 = jnp.concatenate(
        [jnp.mean(v, axis=1, keepdims=True) for v in l4s], axis=0)

    dn = (((0,), (0,)), ((), ()))
    h = (jax.lax.dot_general(m12, w1_ref[0:72, :], dn,
                             preferred_element_type=jnp.float32)
         + jax.lax.dot_general(m3, w1_ref[72:584, :], dn,
                               preferred_element_type=jnp.float32)
         + jax.lax.dot_general(m4, w1_ref[584:4680, :], dn,
                               preferred_element_type=jnp.float32)
         + b1_ref[...])
    h = jnp.maximum(h, 0.0)
    out_ref[...] = (jnp.dot(h, w2_ref[...], preferred_element_type=jnp.float32)
                    + b2_ref[...])[None]


def kernel(x, W1, b1, W2, b2):
    B, T, Cin = x.shape
    C = Cin + 1
    N = T // _L
    H = W1.shape[1]

    t = jnp.linspace(0.0, 1.0, T, dtype=x.dtype)
    path = jnp.concatenate(
        [x, jnp.broadcast_to(t[None, :, None], (B, T, 1)).astype(x.dtype)],
        axis=-1)
    inc = jnp.diff(path, axis=1, prepend=jnp.zeros((B, 1, C), x.dtype))
    inc_t = inc.reshape(B, N, _L, C).transpose(0, 2, 3, 1)

    W1r = jnp.concatenate(
        [W1[:584],
         W1[584:].reshape(C, C, C, C, H).transpose(3, 0, 1, 2, 4)
                 .reshape(C ** 4, H)], axis=0)

    b1_2d = b1.reshape(1, H)
    b2_2d = b2.reshape(1, 1)

    out = pl.pallas_call(
        _blamem_kernel,
        grid=(B,),
        in_specs=[
            pl.BlockSpec((1, _L, C, N), lambda b: (b, 0, 0, 0)),
            pl.BlockSpec(W1.shape, lambda b: (0, 0)),
            pl.BlockSpec((1, H), lambda b: (0, 0)),
            pl.BlockSpec(W2.shape, lambda b: (0, 0)),
            pl.BlockSpec((1, 1), lambda b: (0, 0)),
        ],
        out_specs=pl.BlockSpec((1, 1, 1), lambda b: (b, 0, 0)),
        out_shape=jax.ShapeDtypeStruct((B, 1, 1), jnp.float32),
        compiler_params=pltpu.CompilerParams(
            dimension_semantics=("arbitrary",),
            vmem_limit_bytes=56 * 1024 * 1024,
        ),
    )(inc_t, W1r, b1_2d, W2, b2_2d)
    return out.reshape(B, 1)
